# Initial kernel scaffold; baseline (speedup 1.0000x reference)
import jax
import jax.numpy as jnp
from jax.experimental import pallas as pl


def kernel(x, current_mask, W1, b1, W2):
    raise NotImplementedError("write your pallas kernel here")


# speedup vs baseline: 1.9191x; 1.9191x over previous
import functools

import jax
import jax.numpy as jnp
from jax import lax
from jax.experimental import pallas as pl
from jax.experimental.pallas import tpu as pltpu

_CAPACITY_FACTOR = 0.67
_AUX_LOSS_COEF = 0.001


def _router_body(x_ref, w1_ref, b1_ref, w2t_ref, maskf_ref,
                 mask_out_ref, aux_ref, scores_ref,
                 *, k_keep, n_tiles, m_tile):
    i = pl.program_id(0)

    h = jnp.dot(x_ref[...], w1_ref[...], preferred_element_type=jnp.float32)
    h = h + b1_ref[...]
    h = h * jax.nn.sigmoid(h)
    logits = lax.dot_general(w2t_ref[...], h, (((1,), (1,)), ((), ())),
                             preferred_element_type=jnp.float32)
    s = jax.nn.sigmoid(logits) * maskf_ref[0]
    scores_ref[pl.ds(i, 1), :] = s

    @pl.when(i == n_tiles - 1)
    def _select():
        scores = scores_ref[...]
        bits = lax.bitcast_convert_type(scores, jnp.int32)
        n = n_tiles * m_tile

        def bisect_step(_, lo_hi):
            lo, hi = lo_hi
            mid = lo + lax.shift_right_logical(hi - lo, 1)
            c = jnp.sum((bits > mid).astype(jnp.int32))
            big = c >= k_keep
            return (jnp.where(big, mid + 1, lo), jnp.where(big, hi, mid))

        lo, _ = lax.fori_loop(0, 30, bisect_step,
                              (jnp.int32(0), jnp.int32(0x3F800000)))
        v = lo
        gt = bits > v
        eq = bits == v
        c_gt = jnp.sum(gt.astype(jnp.int32))
        t_ties = k_keep - c_gt

        idx = (lax.broadcasted_iota(jnp.int32, bits.shape, 0) * m_tile
               + lax.broadcasted_iota(jnp.int32, bits.shape, 1))
        eq_i = eq.astype(jnp.int32)

        def tie_step(_, lo_hi):
            lo, hi = lo_hi
            mid = lo + lax.shift_right_logical(hi - lo, 1)
            c = jnp.sum(jnp.where(idx < mid, eq_i, 0))
            small = c < t_ties
            return (jnp.where(small, mid + 1, lo), jnp.where(small, hi, mid))

        m_star, _ = lax.fori_loop(0, 15, tie_step,
                                  (jnp.int32(0), jnp.int32(n)))

        accept = jnp.logical_or(gt, jnp.logical_and(eq, idx < m_star))
        mask_out_ref[...] = accept.astype(jnp.int32)

        v_f = lax.bitcast_convert_type(v, jnp.float32)
        sum_sel = (jnp.sum(jnp.where(gt, scores, 0.0))
                   + t_ties.astype(jnp.float32) * v_f)
        aux_ref[0, 0] = -(sum_sel / jnp.float32(k_keep)) * jnp.float32(_AUX_LOSS_COEF)


def kernel(x, current_mask, W1, b1, W2):
    batch, seq, d_model = x.shape
    d_hidden = W1.shape[1]
    n = batch * seq
    if n == 0:
        return current_mask, jnp.array(0.0, dtype=jnp.float32)
    k_keep = max(1, int(n * _CAPACITY_FACTOR))
    if k_keep >= n:
        return current_mask, jnp.array(0.0, dtype=jnp.float32)

    m_tile = 512
    assert n % m_tile == 0
    n_tiles = n // m_tile

    x2 = x.reshape(n, d_model)
    maskf = current_mask.reshape(n_tiles, 1, m_tile).astype(jnp.float32)
    b1r = b1.reshape(1, d_hidden)
    w2t = W2.reshape(1, d_hidden)

    mask_out, aux = pl.pallas_call(
        functools.partial(_router_body, k_keep=k_keep, n_tiles=n_tiles,
                          m_tile=m_tile),
        grid=(n_tiles,),
        in_specs=[
            pl.BlockSpec((m_tile, d_model), lambda i: (i, 0)),
            pl.BlockSpec((d_model, d_hidden), lambda i: (0, 0)),
            pl.BlockSpec((1, d_hidden), lambda i: (0, 0)),
            pl.BlockSpec((1, d_hidden), lambda i: (0, 0)),
            pl.BlockSpec((1, 1, m_tile), lambda i: (i, 0, 0)),
        ],
        out_specs=[
            pl.BlockSpec((n_tiles, m_tile), lambda i: (0, 0)),
            pl.BlockSpec(memory_space=pltpu.SMEM),
        ],
        out_shape=[
            jax.ShapeDtypeStruct((n_tiles, m_tile), jnp.int32),
            jax.ShapeDtypeStruct((1, 1), jnp.float32),
        ],
        scratch_shapes=[pltpu.VMEM((n_tiles, m_tile), jnp.float32)],
        compiler_params=pltpu.CompilerParams(
            dimension_semantics=("arbitrary",)),
    )(x2, W1, b1r, w2t, maskf)

    new_mask = mask_out.reshape(batch, seq).astype(bool)
    return new_mask, aux[0, 0]


# speedup vs baseline: 2.2416x; 1.1681x over previous
import functools

import jax
import jax.numpy as jnp
from jax import lax
from jax.experimental import pallas as pl
from jax.experimental.pallas import tpu as pltpu

_CAPACITY_FACTOR = 0.67
_AUX_LOSS_COEF = 0.001


def _router_body(x0_ref, x1_ref, x2_ref, x3_ref, w1_ref, b1_ref, w2t_ref,
                 maskf_ref, mask_out_ref, aux_ref, scores_ref,
                 *, k_keep, n_steps, sub_tile):
    i = pl.program_id(0)
    n_rows = 4 * n_steps

    for j, x_ref in enumerate((x0_ref, x1_ref, x2_ref, x3_ref)):
        h = jnp.dot(x_ref[...], w1_ref[...],
                    preferred_element_type=jnp.float32)
        h = h + b1_ref[...]
        h = h * jax.nn.sigmoid(h)
        logits = lax.dot_general(w2t_ref[...], h, (((1,), (1,)), ((), ())),
                                 preferred_element_type=jnp.float32)
        s = jax.nn.sigmoid(logits) * maskf_ref[0][:, j * sub_tile:(j + 1) * sub_tile]
        scores_ref[pl.ds(4 * i + j, 1), :] = s

    @pl.when(i == n_steps - 1)
    def _select():
        scores = scores_ref[...]
        bits = lax.bitcast_convert_type(scores, jnp.int32)
        n = n_rows * sub_tile

        def bisect_step(_, lo_hi):
            lo, hi = lo_hi
            q = lax.shift_right_logical(hi - lo, 2)
            m1 = lo + q
            m2 = lo + 2 * q
            m3 = lo + 3 * q
            c1 = jnp.sum((bits > m1).astype(jnp.int32))
            c2 = jnp.sum((bits > m2).astype(jnp.int32))
            c3 = jnp.sum((bits > m3).astype(jnp.int32))
            lo2 = jnp.where(c3 >= k_keep, m3 + 1,
                  jnp.where(c2 >= k_keep, m2 + 1,
                  jnp.where(c1 >= k_keep, m1 + 1, lo)))
            hi2 = jnp.where(c3 >= k_keep, hi,
                  jnp.where(c2 >= k_keep, m3,
                  jnp.where(c1 >= k_keep, m2, m1)))
            return lo2, hi2

        lo, _ = lax.fori_loop(0, 20, bisect_step,
                              (jnp.int32(0), jnp.int32(0x3F800000)))
        v = lo
        gt = bits > v
        eq = bits == v
        c_gt = jnp.sum(gt.astype(jnp.int32))
        t_ties = k_keep - c_gt

        idx = (lax.broadcasted_iota(jnp.int32, bits.shape, 0) * sub_tile
               + lax.broadcasted_iota(jnp.int32, bits.shape, 1))
        eq_i = eq.astype(jnp.int32)

        def tie_step(_, lo_hi):
            lo, hi = lo_hi
            q = lax.shift_right_logical(hi - lo, 2)
            m1 = lo + q
            m2 = lo + 2 * q
            m3 = lo + 3 * q
            c1 = jnp.sum(jnp.where(idx < m1, eq_i, 0))
            c2 = jnp.sum(jnp.where(idx < m2, eq_i, 0))
            c3 = jnp.sum(jnp.where(idx < m3, eq_i, 0))
            lo2 = jnp.where(c1 < t_ties, m1 + 1, lo)
            lo2 = jnp.where(c2 < t_ties, m2 + 1, lo2)
            lo2 = jnp.where(c3 < t_ties, m3 + 1, lo2)
            hi2 = jnp.where(c1 >= t_ties, m1,
                  jnp.where(c2 >= t_ties, m2,
                  jnp.where(c3 >= t_ties, m3, hi)))
            return lo2, hi2

        m_star, _ = lax.fori_loop(0, 13, tie_step,
                                  (jnp.int32(0), jnp.int32(n)))

        accept = jnp.logical_or(gt, jnp.logical_and(eq, idx < m_star))
        mask_out_ref[...] = accept.astype(jnp.int32)

        v_f = lax.bitcast_convert_type(v, jnp.float32)
        sum_sel = (jnp.sum(jnp.where(gt, scores, 0.0))
                   + t_ties.astype(jnp.float32) * v_f)
        aux_ref[0, 0] = -(sum_sel / jnp.float32(k_keep)) * jnp.float32(_AUX_LOSS_COEF)


def kernel(x, current_mask, W1, b1, W2):
    batch, seq, d_model = x.shape
    d_hidden = W1.shape[1]
    n = batch * seq
    if n == 0:
        return current_mask, jnp.array(0.0, dtype=jnp.float32)
    k_keep = max(1, int(n * _CAPACITY_FACTOR))
    if k_keep >= n:
        return current_mask, jnp.array(0.0, dtype=jnp.float32)

    m_tile = 2048
    sub_tile = m_tile // 4
    assert n % m_tile == 0
    n_steps = n // m_tile
    n_rows = 4 * n_steps

    x2 = x.reshape(n, d_model)
    maskf = current_mask.reshape(n_steps, 1, m_tile).astype(jnp.float32)
    b1r = b1.reshape(1, d_hidden)
    w2t = W2.reshape(1, d_hidden)

    def _x_spec(j):
        return pl.BlockSpec((sub_tile, d_model), lambda i, j=j: (4 * i + j, 0))

    mask_out, aux = pl.pallas_call(
        functools.partial(_router_body, k_keep=k_keep, n_steps=n_steps,
                          sub_tile=sub_tile),
        grid=(n_steps,),
        in_specs=[
            _x_spec(0),
            _x_spec(1),
            _x_spec(2),
            _x_spec(3),
            pl.BlockSpec((d_model, d_hidden), lambda i: (0, 0)),
            pl.BlockSpec((1, d_hidden), lambda i: (0, 0)),
            pl.BlockSpec((1, d_hidden), lambda i: (0, 0)),
            pl.BlockSpec((1, 1, m_tile), lambda i: (i, 0, 0)),
        ],
        out_specs=[
            pl.BlockSpec((n_rows, sub_tile), lambda i: (0, 0)),
            pl.BlockSpec(memory_space=pltpu.SMEM),
        ],
        out_shape=[
            jax.ShapeDtypeStruct((n_rows, sub_tile), jnp.int32),
            jax.ShapeDtypeStruct((1, 1), jnp.float32),
        ],
        scratch_shapes=[pltpu.VMEM((n_rows, sub_tile), jnp.float32)],
        compiler_params=pltpu.CompilerParams(
            dimension_semantics=("arbitrary",)),
    )(x2, x2, x2, x2, W1, b1r, w2t, maskf)

    new_mask = mask_out.reshape(batch, seq).astype(bool)
    return new_mask, aux[0, 0]
